# baseline (speedup 1.0000x reference)
import jax
import jax.numpy as jnp
from jax.experimental import pallas as pl

N = 128
D = 4
HID = 16
STEPS = 4
BP = 16
GRP = 2
SCH = 32


def _blockdiag(W, m, n):
    T = jnp.tile(W, (BP, BP))
    r = jax.lax.broadcasted_iota(jnp.int32, (BP * m, BP * n), 0)
    c = jax.lax.broadcasted_iota(jnp.int32, (BP * m, BP * n), 1)
    return jnp.where(r // m == c // n, T, 0.0)


def _rep(v):
    return jnp.tile(v.reshape(1, -1), (1, BP))


def _swarm_body(xp_ref, We1_ref, be1_ref, We2_ref, be2_ref,
                Wn1_ref, bn1_ref, Wn2_ref, bn2_ref,
                Wd1_ref, bd1_ref, Wd2_ref, bd2_ref,
                Wo_ref, bo_ref, out_ref):
    We1 = We1_ref[...]
    Wsb = _blockdiag(We1[:D], D, HID)
    Wtb = _blockdiag(We1[D:], D, HID)
    be1 = _rep(be1_ref[...])
    W2b = _blockdiag(We2_ref[...], HID, HID)
    be2 = _rep(be2_ref[...])
    nbe2 = -be2
    Wn1b = _blockdiag(Wn1_ref[...], HID, HID)
    bn1 = _rep(bn1_ref[...]) + (N - 1.0) * jnp.dot(
        be2, Wn1b, preferred_element_type=jnp.float32)
    Wn2b = _blockdiag(Wn2_ref[...], HID, HID)
    bn2 = _rep(bn2_ref[...])
    Wd1 = Wd1_ref[...]
    Wd1xb = _blockdiag(Wd1[:D], D, HID)
    Wd1mb = _blockdiag(Wd1[D:], HID, HID)
    bd1 = _rep(bd1_ref[...])
    Wd2b = _blockdiag(Wd2_ref[...], HID, HID)
    bd2 = _rep(bd2_ref[...])
    Wob = _blockdiag(Wo_ref[...], HID, D)
    bo = _rep(bo_ref[...])
    W2bf = W2b.astype(jnp.bfloat16)
    nbe2bf = nbe2.astype(jnp.bfloat16)

    L = BP * HID
    dot = lambda a, b: jnp.dot(a, b, preferred_element_type=jnp.float32)

    xs = [xp_ref[g] for g in range(GRP)]
    for step in range(STEPS):
        for g in range(GRP):
            xp = xs[g]
            Arow = dot(xp, Wsb)
            Crow = dot(xp, Wtb) + be1
            Abf = Arow.astype(jnp.bfloat16)
            Cbf = Crow.astype(jnp.bfloat16)
            S = jnp.zeros((N, L), jnp.float32)
            for s0 in range(0, N, SCH):
                Hc = jax.nn.relu(
                    Abf[s0:s0 + SCH, None, :] + Cbf[None, :, :])
                Yc = dot(Hc.reshape(SCH * N, L), W2bf)
                v = jnp.maximum(Yc.astype(jnp.bfloat16), nbe2bf)
                half = SCH // 2
                while half >= 1:
                    v = (v.reshape(2, half * N, L)[0]
                         + v.reshape(2, half * N, L)[1])
                    half //= 2
                S = S + v.astype(jnp.float32)
            D2 = jnp.maximum(dot(jax.nn.relu(Abf + Cbf), W2bf), nbe2)
            msg = S - D2
            msg = jax.nn.relu(dot(msg, Wn1b) + bn1)
            msg = jax.nn.relu(dot(msg, Wn2b) + bn2)
            z = jax.nn.relu(dot(xp, Wd1xb) + dot(msg, Wd1mb) + bd1)
            z = jax.nn.relu(dot(z, Wd2b) + bd2)
            xp = dot(z, Wob) + bo + xp
            out_ref[g, step] = xp
            xs[g] = xp


def kernel(time_segs, We1, be1, We2, be2, Wn1, bn1, Wn2, bn2,
           Wd1, bd1, Wd2, bd2, Wo, bo):
    B = time_segs.shape[0]
    G = B // BP
    xp = jnp.transpose(time_segs.reshape(G, BP, N, D), (0, 2, 1, 3))
    xp = xp.reshape(G, N, BP * D)

    w_spec = lambda shape: pl.BlockSpec(shape, lambda b: (0,) * len(shape))

    out = pl.pallas_call(
        _swarm_body,
        grid=(G // GRP,),
        in_specs=[
            pl.BlockSpec((GRP, N, BP * D), lambda b: (b, 0, 0)),
            w_spec((2 * D, HID)), w_spec((1, HID)),
            w_spec((HID, HID)), w_spec((1, HID)),
            w_spec((HID, HID)), w_spec((1, HID)),
            w_spec((HID, HID)), w_spec((1, HID)),
            w_spec((D + HID, HID)), w_spec((1, HID)),
            w_spec((HID, HID)), w_spec((1, HID)),
            w_spec((HID, D)), w_spec((1, D)),
        ],
        out_specs=pl.BlockSpec((GRP, STEPS, N, BP * D),
                               lambda b: (b, 0, 0, 0)),
        out_shape=jax.ShapeDtypeStruct((G, STEPS, N, BP * D), jnp.float32),
    )(xp, We1, be1.reshape(1, HID), We2, be2.reshape(1, HID),
      Wn1, bn1.reshape(1, HID), Wn2, bn2.reshape(1, HID),
      Wd1, bd1.reshape(1, HID), Wd2, bd2.reshape(1, HID),
      Wo, bo.reshape(1, D))

    out = out.reshape(G, STEPS, N, BP, D)
    out = jnp.transpose(out, (0, 3, 1, 2, 4)).reshape(B, STEPS, N, D)
    return out


# speedup vs baseline: 1.0582x; 1.0582x over previous
import jax
import jax.numpy as jnp
from jax.experimental import pallas as pl

N = 128
D = 4
HID = 16
STEPS = 4
BP = 16
GRP = 2
SCH = 64


def _blockdiag(W, m, n):
    T = jnp.tile(W, (BP, BP))
    r = jax.lax.broadcasted_iota(jnp.int32, (BP * m, BP * n), 0)
    c = jax.lax.broadcasted_iota(jnp.int32, (BP * m, BP * n), 1)
    return jnp.where(r // m == c // n, T, 0.0)


def _rep(v):
    return jnp.tile(v.reshape(1, -1), (1, BP))


def _swarm_body(xp_ref, We1_ref, be1_ref, We2_ref, be2_ref,
                Wn1_ref, bn1_ref, Wn2_ref, bn2_ref,
                Wd1_ref, bd1_ref, Wd2_ref, bd2_ref,
                Wo_ref, bo_ref, out_ref):
    We1 = We1_ref[...]
    Wsb = _blockdiag(We1[:D], D, HID)
    Wtb = _blockdiag(We1[D:], D, HID)
    be1 = _rep(be1_ref[...])
    W2b = _blockdiag(We2_ref[...], HID, HID)
    be2 = _rep(be2_ref[...])
    nbe2 = -be2
    Wn1b = _blockdiag(Wn1_ref[...], HID, HID)
    bn1 = _rep(bn1_ref[...]) + (N - 1.0) * jnp.dot(
        be2, Wn1b, preferred_element_type=jnp.float32)
    Wn2b = _blockdiag(Wn2_ref[...], HID, HID)
    bn2 = _rep(bn2_ref[...])
    Wd1 = Wd1_ref[...]
    Wd1xb = _blockdiag(Wd1[:D], D, HID)
    Wd1mb = _blockdiag(Wd1[D:], HID, HID)
    bd1 = _rep(bd1_ref[...])
    Wd2b = _blockdiag(Wd2_ref[...], HID, HID)
    bd2 = _rep(bd2_ref[...])
    Wob = _blockdiag(Wo_ref[...], HID, D)
    bo = _rep(bo_ref[...])
    W2bf = W2b.astype(jnp.bfloat16)

    L = BP * HID
    dot = lambda a, b: jnp.dot(a, b, preferred_element_type=jnp.float32)

    xs = [xp_ref[g] for g in range(GRP)]
    for step in range(STEPS):
        for g in range(GRP):
            xp = xs[g]
            Arow = dot(xp, Wsb)
            Crow = dot(xp, Wtb) + be1
            Abf = Arow.astype(jnp.bfloat16)
            Cbf = Crow.astype(jnp.bfloat16)
            S = jnp.zeros((N, L), jnp.float32)
            for s0 in range(0, N, SCH):
                Hc = jax.nn.relu(
                    Abf[s0:s0 + SCH, None, :] + Cbf[None, :, :])
                Yc = dot(Hc.reshape(SCH * N, L), W2bf)
                S = S + jnp.sum(
                    jnp.maximum(Yc, nbe2).reshape(SCH, N, L), axis=0)
            D2 = jnp.maximum(dot(jax.nn.relu(Abf + Cbf), W2bf), nbe2)
            msg = S - D2
            msg = jax.nn.relu(dot(msg, Wn1b) + bn1)
            msg = jax.nn.relu(dot(msg, Wn2b) + bn2)
            z = jax.nn.relu(dot(xp, Wd1xb) + dot(msg, Wd1mb) + bd1)
            z = jax.nn.relu(dot(z, Wd2b) + bd2)
            xp = dot(z, Wob) + bo + xp
            out_ref[g, step] = xp
            xs[g] = xp


def kernel(time_segs, We1, be1, We2, be2, Wn1, bn1, Wn2, bn2,
           Wd1, bd1, Wd2, bd2, Wo, bo):
    B = time_segs.shape[0]
    G = B // BP
    xp = jnp.transpose(time_segs.reshape(G, BP, N, D), (0, 2, 1, 3))
    xp = xp.reshape(G, N, BP * D)

    w_spec = lambda shape: pl.BlockSpec(shape, lambda b: (0,) * len(shape))

    out = pl.pallas_call(
        _swarm_body,
        grid=(G // GRP,),
        in_specs=[
            pl.BlockSpec((GRP, N, BP * D), lambda b: (b, 0, 0)),
            w_spec((2 * D, HID)), w_spec((1, HID)),
            w_spec((HID, HID)), w_spec((1, HID)),
            w_spec((HID, HID)), w_spec((1, HID)),
            w_spec((HID, HID)), w_spec((1, HID)),
            w_spec((D + HID, HID)), w_spec((1, HID)),
            w_spec((HID, HID)), w_spec((1, HID)),
            w_spec((HID, D)), w_spec((1, D)),
        ],
        out_specs=pl.BlockSpec((GRP, STEPS, N, BP * D),
                               lambda b: (b, 0, 0, 0)),
        out_shape=jax.ShapeDtypeStruct((G, STEPS, N, BP * D), jnp.float32),
    )(xp, We1, be1.reshape(1, HID), We2, be2.reshape(1, HID),
      Wn1, bn1.reshape(1, HID), Wn2, bn2.reshape(1, HID),
      Wd1, bd1.reshape(1, HID), Wd2, bd2.reshape(1, HID),
      Wo, bo.reshape(1, D))

    out = out.reshape(G, STEPS, N, BP, D)
    out = jnp.transpose(out, (0, 3, 1, 2, 4)).reshape(B, STEPS, N, D)
    return out
